# baseline (speedup 1.0000x reference)
import functools

import jax
import jax.numpy as jnp
from jax import lax
from jax.experimental import pallas as pl
from jax.experimental.pallas import tpu as pltpu
from jax.experimental.pallas import tpu_sc as plsc

N = 10000
K = 32
D = 128
OUT = 128

NC = 2
NS = 16
NW = NC * NS
LANES = 16

B = 8
G = N // B
NB = -(-G // NW)
NB2 = -(-NB // 2)


def _sc_aggregate(h_nn):
    mesh = plsc.VectorSubcoreMesh(core_axis_name="c", subcore_axis_name="s")

    @functools.partial(
        pl.kernel,
        mesh=mesh,
        out_type=jax.ShapeDtypeStruct((N, D), jnp.float32),
        scratch_types=[
            pltpu.VMEM((B, K, D), jnp.float32),
            pltpu.VMEM((B, K, D), jnp.float32),
            pltpu.VMEM((B, D), jnp.float32),
            pltpu.SemaphoreType.DMA,
            pltpu.SemaphoreType.DMA,
        ],
    )
    def aggr_kernel(hnn_hbm, out_hbm, buf0, buf1, acc, sem0, sem1):
        wid = lax.axis_index("s") * NC + lax.axis_index("c")
        start_g = (wid * G) // NW
        end_g = ((wid + 1) * G) // NW

        def base_of(i):
            return jnp.minimum(start_g + i, end_g - 1) * B

        def start_copy(i, buf, sem):
            pltpu.async_copy(hnn_hbm.at[pl.ds(base_of(i), B)], buf, sem)

        def wait_copy(buf, sem):
            pltpu.make_async_copy(hnn_hbm.at[pl.ds(0, B)], buf, sem).wait()

        def compute(i, buf):
            for bb in range(B):
                for blk in range(D // LANES):
                    col = pl.ds(blk * LANES, LANES)
                    acc_v = buf[bb, 0, col]
                    for k in range(1, K):
                        acc_v = acc_v + buf[bb, k, col]
                    acc[bb, col] = acc_v
            pltpu.sync_copy(acc, out_hbm.at[pl.ds(base_of(i), B)])

        start_copy(0, buf0, sem0)

        def body(t, carry):
            i0 = 2 * t
            wait_copy(buf0, sem0)
            start_copy(i0 + 1, buf1, sem1)
            compute(i0, buf0)
            wait_copy(buf1, sem1)
            start_copy(i0 + 2, buf0, sem0)
            compute(i0 + 1, buf1)
            return carry

        lax.fori_loop(0, NB2, body, 0, unroll=False)
        wait_copy(buf0, sem0)

    return aggr_kernel(h_nn)


BLOCK_M = 1000


def _tc_body(h_ref, aggr_ref, w1_ref, w2_ref, b_ref, o_ref):
    o_ref[...] = (
        jnp.dot(h_ref[...], w1_ref[...], preferred_element_type=jnp.float32)
        + jnp.dot(aggr_ref[...], w2_ref[...], preferred_element_type=jnp.float32)
        + b_ref[...]
    )


def _tc_linear(h, aggr, W, b):
    w1t = W[:, :D].T
    w2t = W[:, D:].T
    b2 = b.reshape(1, OUT)
    return pl.pallas_call(
        _tc_body,
        grid=(N // BLOCK_M,),
        in_specs=[
            pl.BlockSpec((BLOCK_M, D), lambda i: (i, 0)),
            pl.BlockSpec((BLOCK_M, D), lambda i: (i, 0)),
            pl.BlockSpec((D, OUT), lambda i: (0, 0)),
            pl.BlockSpec((D, OUT), lambda i: (0, 0)),
            pl.BlockSpec((1, OUT), lambda i: (0, 0)),
        ],
        out_specs=pl.BlockSpec((BLOCK_M, OUT), lambda i: (i, 0)),
        out_shape=jax.ShapeDtypeStruct((N, OUT), jnp.float32),
    )(h, aggr, w1t, w2t, b2)


def kernel(h, h_nn, W, b):
    aggr = _sc_aggregate(h_nn)
    return _tc_linear(h, aggr, W, b)


# speedup vs baseline: 1.8413x; 1.8413x over previous
import functools

import jax
import jax.numpy as jnp
from jax import lax
from jax.experimental import pallas as pl
from jax.experimental.pallas import tpu as pltpu
from jax.experimental.pallas import tpu_sc as plsc

N = 10000
K = 32
D = 128
OUT = 128

NC = 2
NS = 16
LANES = 16

B = 8
BK = B * K
G = N // B
GC = G // NC
GW = GC // NS
ROWS_W = GW * B
ZROWS = 104


def _sc_aggregate(h_nn2d):
    mesh = plsc.VectorSubcoreMesh(core_axis_name="c", subcore_axis_name="s")

    @functools.partial(
        pl.kernel,
        mesh=mesh,
        out_type=jax.ShapeDtypeStruct((N, D), jnp.float32),
        scratch_types=[
            pltpu.VMEM((BK, D), jnp.float32),
            pltpu.VMEM((BK, D), jnp.float32),
            pltpu.VMEM((BK,), jnp.int32),
            pltpu.VMEM((BK,), jnp.int32),
            pltpu.VMEM((ZROWS, D), jnp.float32),
            pltpu.VMEM_SHARED((GC * B, D), jnp.float32),
            pltpu.SemaphoreType.DMA,
            pltpu.SemaphoreType.DMA,
        ],
    )
    def aggr_kernel(hnn_hbm, out_hbm, buf0, buf1, idx0, idx1, zbuf, acc,
                    si0, si1):
        c = lax.axis_index("c")
        s = lax.axis_index("s")
        start_g = c * GC + s * GW
        rel_row = s * ROWS_W

        def start_in(i, buf, sem):
            g = start_g + i
            pltpu.async_copy(hnn_hbm.at[pl.ds(g * BK, BK)], buf, sem)

        def wait_in(buf, sem):
            pltpu.make_async_copy(hnn_hbm.at[pl.ds(0, BK)], buf, sem).wait()

        def fill_idx(idx, i):
            node0 = rel_row + i * B
            for v in range(BK // LANES):
                idx[pl.ds(v * LANES, LANES)] = jnp.full(
                    (LANES,), node0 + v // 2, jnp.int32)

        def scatter_add(buf, idx):
            pltpu.sync_copy(buf, acc.at[idx], add=True)

        for r in range(ZROWS):
            for cb in range(D // LANES):
                zbuf[r, pl.ds(cb * LANES, LANES)] = jnp.zeros(
                    (LANES,), jnp.float32)
        for z in range(ROWS_W // ZROWS):
            pltpu.sync_copy(zbuf, acc.at[pl.ds(rel_row + z * ZROWS, ZROWS)])

        @pl.when(s == NS - 1)
        def _zero_tail():
            pltpu.sync_copy(zbuf.at[pl.ds(0, B)],
                            acc.at[pl.ds(rel_row + ROWS_W, B)])

        start_in(0, buf0, si0)

        def body(t, carry):
            i0 = 2 * t
            wait_in(buf0, si0)
            start_in(i0 + 1, buf1, si1)
            fill_idx(idx0, i0)
            scatter_add(buf0, idx0)
            wait_in(buf1, si1)
            start_in(i0 + 2, buf0, si0)
            fill_idx(idx1, i0 + 1)
            scatter_add(buf1, idx1)
            return carry

        lax.fori_loop(0, (GW - 1) // 2, body, 0, unroll=False)
        wait_in(buf0, si0)
        fill_idx(idx0, GW - 1)
        scatter_add(buf0, idx0)

        @pl.when(s == NS - 1)
        def _tail_group():
            pltpu.sync_copy(hnn_hbm.at[pl.ds((start_g + GW) * BK, BK)], buf1)
            fill_idx(idx1, GW)
            scatter_add(buf1, idx1)

        out0 = c * GC * B + s * ROWS_W
        pltpu.sync_copy(acc.at[pl.ds(rel_row, ROWS_W)],
                        out_hbm.at[pl.ds(out0, ROWS_W)])

        @pl.when(s == NS - 1)
        def _out_tail():
            pltpu.sync_copy(acc.at[pl.ds(rel_row + ROWS_W, B)],
                            out_hbm.at[pl.ds(out0 + ROWS_W, B)])

    return aggr_kernel(h_nn2d)


BLOCK_M = 1000


def _tc_body(h_ref, aggr_ref, w1_ref, w2_ref, b_ref, o_ref):
    o_ref[...] = (
        jnp.dot(h_ref[...], w1_ref[...], preferred_element_type=jnp.float32)
        + jnp.dot(aggr_ref[...], w2_ref[...], preferred_element_type=jnp.float32)
        + b_ref[...]
    )


def _tc_linear(h, aggr, W, b):
    w1t = W[:, :D].T
    w2t = W[:, D:].T
    b2 = b.reshape(1, OUT)
    return pl.pallas_call(
        _tc_body,
        grid=(N // BLOCK_M,),
        in_specs=[
            pl.BlockSpec((BLOCK_M, D), lambda i: (i, 0)),
            pl.BlockSpec((BLOCK_M, D), lambda i: (i, 0)),
            pl.BlockSpec((D, OUT), lambda i: (0, 0)),
            pl.BlockSpec((D, OUT), lambda i: (0, 0)),
            pl.BlockSpec((1, OUT), lambda i: (0, 0)),
        ],
        out_specs=pl.BlockSpec((BLOCK_M, OUT), lambda i: (i, 0)),
        out_shape=jax.ShapeDtypeStruct((N, OUT), jnp.float32),
    )(h, aggr, w1t, w2t, b2)


def kernel(h, h_nn, W, b):
    aggr = _sc_aggregate(h_nn.reshape(N * K, D))
    return _tc_linear(h, aggr, W, b)


# speedup vs baseline: 3.4692x; 1.8841x over previous
import functools

import jax
import jax.numpy as jnp
from jax import lax
from jax.experimental import pallas as pl
from jax.experimental.pallas import tpu as pltpu
from jax.experimental.pallas import tpu_sc as plsc

N = 10000
K = 32
D = 128
OUT = 128

BLOCK_M = 400
N_SC = 2800
N_TC = N - N_SC

NC = 2
NS = 16
LANES = 16

B = 8
BK = B * K
TG = N_SC // B
GC = TG // NC
MAXB = -(-GC // NS)
ZROWS = (GC // NS) * B


def _sc_aggregate(h_nn2d):
    mesh = plsc.VectorSubcoreMesh(core_axis_name="c", subcore_axis_name="s")

    @functools.partial(
        pl.kernel,
        mesh=mesh,
        out_type=jax.ShapeDtypeStruct((N_SC, D), jnp.float32),
        scratch_types=[
            pltpu.VMEM((BK, D), jnp.float32),
            pltpu.VMEM((BK, D), jnp.float32),
            pltpu.VMEM((BK,), jnp.int32),
            pltpu.VMEM((BK,), jnp.int32),
            pltpu.VMEM((ZROWS, D), jnp.float32),
            pltpu.VMEM_SHARED((GC * B, D), jnp.float32),
            pltpu.SemaphoreType.DMA,
            pltpu.SemaphoreType.DMA,
        ],
    )
    def aggr_kernel(hnn_hbm, out_hbm, buf0, buf1, idx0, idx1, zbuf, acc,
                    si0, si1):
        c = lax.axis_index("c")
        s = lax.axis_index("s")
        start_rel_g = (s * GC) // NS
        end_rel_g = ((s + 1) * GC) // NS
        count = end_rel_g - start_rel_g
        rel_row = start_rel_g * B

        def start_in(i, buf, sem):
            g = c * GC + jnp.minimum(start_rel_g + i, end_rel_g - 1)
            pltpu.async_copy(
                hnn_hbm.at[pl.ds((N_TC // B + g) * BK, BK)], buf, sem)

        def wait_in(buf, sem):
            pltpu.make_async_copy(hnn_hbm.at[pl.ds(0, BK)], buf, sem).wait()

        def fill_idx(idx, i):
            node0 = rel_row + i * B
            for v in range(BK // LANES):
                idx[pl.ds(v * LANES, LANES)] = jnp.full(
                    (LANES,), node0 + v // 2, jnp.int32)

        def scatter_add(buf, idx):
            pltpu.sync_copy(buf, acc.at[idx], add=True)

        for r in range(ZROWS):
            for cb in range(D // LANES):
                zbuf[r, pl.ds(cb * LANES, LANES)] = jnp.zeros(
                    (LANES,), jnp.float32)
        pltpu.sync_copy(zbuf, acc.at[pl.ds(rel_row, ZROWS)])

        @pl.when(count == MAXB)
        def _zero_tail():
            pltpu.sync_copy(zbuf.at[pl.ds(0, B)],
                            acc.at[pl.ds(rel_row + ZROWS, B)])

        start_in(0, buf0, si0)

        def body(t, carry):
            i0 = 2 * t
            wait_in(buf0, si0)
            start_in(i0 + 1, buf1, si1)
            fill_idx(idx0, i0)
            scatter_add(buf0, idx0)
            wait_in(buf1, si1)
            start_in(i0 + 2, buf0, si0)
            fill_idx(idx1, i0 + 1)
            scatter_add(buf1, idx1)
            return carry

        lax.fori_loop(0, (MAXB - 1) // 2, body, 0, unroll=False)
        wait_in(buf0, si0)

        @pl.when(count == MAXB)
        def _tail_group():
            fill_idx(idx0, MAXB - 1)
            scatter_add(buf0, idx0)

        out0 = c * GC * B + rel_row
        pltpu.sync_copy(acc.at[pl.ds(rel_row, ZROWS)],
                        out_hbm.at[pl.ds(out0, ZROWS)])

        @pl.when(count == MAXB)
        def _out_tail():
            pltpu.sync_copy(acc.at[pl.ds(rel_row + ZROWS, B)],
                            out_hbm.at[pl.ds(out0 + ZROWS, B)])

    return aggr_kernel(h_nn2d)


def _tc_head_body(h_ref, hnn_ref, w1_ref, w2_ref, b_ref, o_ref):
    aggr = jnp.sum(hnn_ref[...], axis=1)
    o_ref[...] = (
        jnp.dot(h_ref[...], w1_ref[...], preferred_element_type=jnp.float32)
        + jnp.dot(aggr, w2_ref[...], preferred_element_type=jnp.float32)
        + b_ref[...]
    )


def _tc_tail_body(h_ref, aggr_ref, w1_ref, w2_ref, b_ref, o_in_ref, o_ref):
    del o_in_ref
    o_ref[...] = (
        jnp.dot(h_ref[...], w1_ref[...], preferred_element_type=jnp.float32)
        + jnp.dot(aggr_ref[...], w2_ref[...], preferred_element_type=jnp.float32)
        + b_ref[...]
    )


def kernel(h, h_nn, W, b):
    w1t = W[:, :D].T
    w2t = W[:, D:].T
    b2 = b.reshape(1, OUT)

    aggr_sc = _sc_aggregate(h_nn.reshape(N * K, D))

    out_head = pl.pallas_call(
        _tc_head_body,
        grid=(N_TC // BLOCK_M,),
        in_specs=[
            pl.BlockSpec((BLOCK_M, D), lambda i: (i, 0)),
            pl.BlockSpec((BLOCK_M, K, D), lambda i: (i, 0, 0)),
            pl.BlockSpec((D, OUT), lambda i: (0, 0)),
            pl.BlockSpec((D, OUT), lambda i: (0, 0)),
            pl.BlockSpec((1, OUT), lambda i: (0, 0)),
        ],
        out_specs=pl.BlockSpec((BLOCK_M, OUT), lambda i: (i, 0)),
        out_shape=jax.ShapeDtypeStruct((N, OUT), jnp.float32),
    )(h, h_nn, w1t, w2t, b2)

    tc0 = N_TC // BLOCK_M
    return pl.pallas_call(
        _tc_tail_body,
        grid=(N_SC // BLOCK_M,),
        in_specs=[
            pl.BlockSpec((BLOCK_M, D), lambda i: (i + tc0, 0)),
            pl.BlockSpec((BLOCK_M, D), lambda i: (i, 0)),
            pl.BlockSpec((D, OUT), lambda i: (0, 0)),
            pl.BlockSpec((D, OUT), lambda i: (0, 0)),
            pl.BlockSpec((1, OUT), lambda i: (0, 0)),
            pl.BlockSpec((BLOCK_M, OUT), lambda i: (i + tc0, 0)),
        ],
        out_specs=pl.BlockSpec((BLOCK_M, OUT), lambda i: (i + tc0, 0)),
        out_shape=jax.ShapeDtypeStruct((N, OUT), jnp.float32),
        input_output_aliases={5: 0},
    )(h, aggr_sc, w1t, w2t, b2, out_head)
